# baseline (speedup 1.0000x reference)
import functools

import jax
import jax.numpy as jnp
from jax import lax
from jax.experimental import pallas as pl
from jax.experimental.pallas import tpu as pltpu
from jax.experimental.pallas import tpu_sc as plsc

_NC = 2
_NS = 16


def _pre_project(x, pos, Wg1, Wp1):
    N, IN = x.shape
    EMB = Wg1.shape[1]
    BLK = 1000

    def body(x_ref, p_ref, wg_ref, wp_ref, yg_ref, yp_ref):
        xb = x_ref[...]
        pb = p_ref[...]
        yg_ref[...] = (
            jnp.dot(xb, wg_ref[:IN, :], preferred_element_type=jnp.float32)
            + jnp.dot(pb, wg_ref[IN:, :], preferred_element_type=jnp.float32)
        )
        yp_ref[...] = jnp.dot(pb, wp_ref[...], preferred_element_type=jnp.float32)

    return pl.pallas_call(
        body,
        grid=(N // BLK,),
        in_specs=[
            pl.BlockSpec((BLK, IN), lambda i: (i, 0)),
            pl.BlockSpec((BLK, pos.shape[1]), lambda i: (i, 0)),
            pl.BlockSpec(Wg1.shape, lambda i: (0, 0)),
            pl.BlockSpec(Wp1.shape, lambda i: (0, 0)),
        ],
        out_specs=[
            pl.BlockSpec((BLK, EMB), lambda i: (i, 0)),
            pl.BlockSpec((BLK, EMB), lambda i: (i, 0)),
        ],
        out_shape=[
            jax.ShapeDtypeStruct((N, EMB), jnp.float32),
            jax.ShapeDtypeStruct((N, EMB), jnp.float32),
        ],
    )(x, pos, Wg1, Wp1)


def _segment_sums(yg, yp, src, dst):
    N, D = yg.shape
    E = src.shape[0]
    K = 2000
    NB = E // K
    G = 128
    ROWS_PT = (N // _NS) // 8 * 8
    REM = N - ROWS_PT * _NS
    AR = ROWS_PT + REM
    GR = AR
    TRASH = K + G
    L = 16

    mesh = plsc.VectorSubcoreMesh(core_axis_name="c", subcore_axis_name="s")

    @functools.partial(
        pl.kernel,
        mesh=mesh,
        compiler_params=pltpu.CompilerParams(needs_layout_passes=False),
        out_type=[
            jax.ShapeDtypeStruct((N, D), jnp.float32),
            jax.ShapeDtypeStruct((N, D), jnp.float32),
        ],
        scratch_types=[
            pltpu.VMEM((K,), jnp.int32),
            pltpu.VMEM((K,), jnp.int32),
            pltpu.VMEM((K + G + 8,), jnp.int32),
            pltpu.VMEM((K + G + 8,), jnp.int32),
            pltpu.VMEM((G,), jnp.int32),
            pltpu.VMEM((G, D), jnp.float32),
            pltpu.VMEM((AR + 8, D), jnp.float32),
            pltpu.SemaphoreType.DMA,
        ],
    )
    def sck(yg_hbm, yp_hbm, src_hbm, dst_hbm, og_hbm, op_hbm,
            src_v, dst_v, csrc_v, cdst_v, crow_v, rows_v, acc_v, sem):
        cid = lax.axis_index("c")
        sid = lax.axis_index("s")
        lo = sid * ROWS_PT
        hi = jnp.where(sid == _NS - 1, N, lo + ROWS_PT)

        def zero_row(r, carry):
            for cc in range(D // L):
                acc_v[r, pl.ds(cc * L, L)] = jnp.zeros((L,), jnp.float32)
            return carry

        lax.fori_loop(0, AR + 8, zero_row, 0)

        def run(y_hbm, out_hbm):
            def block(b, carry):
                pltpu.sync_copy(src_hbm.at[pl.ds(b * K, K)], src_v)
                pltpu.sync_copy(dst_hbm.at[pl.ds(b * K, K)], dst_v)

                lov = jnp.full((L,), lo, jnp.int32)
                hiv = jnp.full((L,), hi, jnp.int32)
                trashv = jnp.full((L,), TRASH, jnp.int32)

                def filt(j, cnt):
                    d = dst_v[pl.ds(j * L, L)]
                    s = src_v[pl.ds(j * L, L)]
                    m = (d >= lov) & (d < hiv)
                    mi = jnp.where(m, 1, 0)
                    cum = plsc.cumsum(mi)
                    cntv = jnp.full((L,), cnt, jnp.int32)
                    pos = jnp.where(m, cntv + cum - 1, trashv)
                    plsc.store_scatter(csrc_v, [pos], s)
                    plsc.store_scatter(cdst_v, [pos], d - lov)
                    return cnt + jnp.sum(mi)

                cnt = lax.fori_loop(0, K // L, filt, 0)

                for jj in range(G // L):
                    csrc_v[pl.ds(cnt + jj * L, L)] = jnp.zeros((L,), jnp.int32)
                    cdst_v[pl.ds(cnt + jj * L, L)] = jnp.full((L,), GR, jnp.int32)

                def batch(bb, carry):
                    pltpu.async_copy(
                        y_hbm.at[csrc_v.at[pl.ds(bb * G, G)]], rows_v, sem
                    ).wait()

                    def grp(j, c2):
                        dv = cdst_v[pl.ds(bb * G + j * L, L)]
                        for jj in range(L):
                            dloc = dv[jj]
                            e = j * L + jj
                            for cc in range(D // L):
                                plsc.addupdate(
                                    acc_v.at[dloc, pl.ds(cc * L, L)],
                                    rows_v[e, pl.ds(cc * L, L)],
                                )
                        return c2

                    lax.fori_loop(0, G // L, grp, 0)
                    return carry

                nb = (cnt + G - 1) // G
                lax.fori_loop(0, nb, batch, 0)
                return carry

            lax.fori_loop(0, NB, block, 0)

            pltpu.sync_copy(
                acc_v.at[pl.ds(0, ROWS_PT)],
                out_hbm.at[pl.ds(sid * ROWS_PT, ROWS_PT)],
            )

            @pl.when(sid == _NS - 1)
            def _():
                pltpu.sync_copy(
                    acc_v.at[pl.ds(ROWS_PT, REM)],
                    out_hbm.at[pl.ds(_NS * ROWS_PT, REM)],
                )

        @pl.when(cid == 0)
        def _():
            run(yg_hbm, og_hbm)

        @pl.when(cid == 1)
        def _():
            run(yp_hbm, op_hbm)

    return sck(yg, yp, src, dst)


def _ln_rows(h, g, b):
    mu = jnp.mean(h, axis=-1, keepdims=True)
    var = jnp.mean((h - mu) * (h - mu), axis=-1, keepdims=True)
    return (h - mu) * jax.lax.rsqrt(var + 1e-5) * g + b


def _epilogue(yg, agg_g, yp, agg_p, x, sg, sp, bg1, lng_g, lng_b, Wg2, bg2,
              bn_g, bn_b, bp1, lnp_g, lnp_b, Wp2, bp2, bnp_g, bnp_b):
    N, D = yg.shape
    BLK = 1000

    def body(sg_ref, sp_ref, yg_ref, ag_ref, yp_ref, ap_ref, x_ref,
             bg1_ref, lng_g_ref, lng_b_ref, wg2_ref, bg2_ref, bn_g_ref, bn_b_ref,
             bp1_ref, lnp_g_ref, lnp_b_ref, wp2_ref, bp2_ref, bnp_g_ref, bnp_b_ref,
             h_ref, p_ref):
        hg = sg_ref[0, 0] * yg_ref[...] + ag_ref[...] + bg1_ref[...]
        hg = jax.nn.relu(_ln_rows(hg, lng_g_ref[...], lng_b_ref[...]))
        hg = jnp.dot(hg, wg2_ref[...], preferred_element_type=jnp.float32) + bg2_ref[...]
        hg = jax.nn.relu(_ln_rows(hg, bn_g_ref[...], bn_b_ref[...]))
        h_ref[...] = hg + x_ref[...]

        hp = sp_ref[0, 0] * yp_ref[...] + ap_ref[...] + bp1_ref[...]
        hp = jax.nn.relu(_ln_rows(hp, lnp_g_ref[...], lnp_b_ref[...]))
        hp = jnp.dot(hp, wp2_ref[...], preferred_element_type=jnp.float32) + bp2_ref[...]
        p_ref[...] = jax.nn.relu(_ln_rows(hp, bnp_g_ref[...], bnp_b_ref[...]))

    row = lambda a: a.reshape(1, D)
    vec_spec = pl.BlockSpec((1, D), lambda i: (0, 0))
    blk_spec = pl.BlockSpec((BLK, D), lambda i: (i, 0))
    mat_spec = pl.BlockSpec((D, D), lambda i: (0, 0))
    smem_spec = pl.BlockSpec(memory_space=pltpu.SMEM)

    return pl.pallas_call(
        body,
        grid=(N // BLK,),
        in_specs=[smem_spec, smem_spec,
                  blk_spec, blk_spec, blk_spec, blk_spec, blk_spec,
                  vec_spec, vec_spec, vec_spec, mat_spec, vec_spec, vec_spec, vec_spec,
                  vec_spec, vec_spec, vec_spec, mat_spec, vec_spec, vec_spec, vec_spec],
        out_specs=[blk_spec, blk_spec],
        out_shape=[
            jax.ShapeDtypeStruct((N, D), jnp.float32),
            jax.ShapeDtypeStruct((N, D), jnp.float32),
        ],
    )(sg.reshape(1, 1), sp.reshape(1, 1),
      yg, agg_g, yp, agg_p, x,
      row(bg1), row(lng_g), row(lng_b), Wg2, row(bg2), row(bn_g), row(bn_b),
      row(bp1), row(lnp_g), row(lnp_b), Wp2, row(bp2), row(bnp_g), row(bnp_b))


def kernel(x, pos_embeddings, eps_g, Wg1, bg1, lng_g, lng_b, Wg2, bg2, bn_g,
           bn_b, eps_p, Wp1, bp1, lnp_g, lnp_b, Wp2, bp2, bnp_g, bnp_b,
           edge_index):
    yg, yp = _pre_project(x, pos_embeddings, Wg1, Wp1)
    src = edge_index[0]
    dst = edge_index[1]
    agg_g, agg_p = _segment_sums(yg, yp, src, dst)
    h, p = _epilogue(yg, agg_g, yp, agg_p, x,
                     1.0 + eps_g, 1.0 + eps_p,
                     bg1, lng_g, lng_b, Wg2, bg2, bn_g, bn_b,
                     bp1, lnp_g, lnp_b, Wp2, bp2, bnp_g, bnp_b)
    return (h, p)


# speedup vs baseline: 7.1445x; 7.1445x over previous
import functools

import jax
import jax.numpy as jnp
from jax import lax
from jax.experimental import pallas as pl
from jax.experimental.pallas import tpu as pltpu
from jax.experimental.pallas import tpu_sc as plsc

_NC = 2
_NS = 16


def _pre_project(x, pos, Wg1, Wp1):
    N, IN = x.shape
    EMB = Wg1.shape[1]
    BLK = 1000

    def body(x_ref, p_ref, wg_ref, wp_ref, yg_ref, yp_ref):
        xb = x_ref[...]
        pb = p_ref[...]
        yg_ref[...] = (
            jnp.dot(xb, wg_ref[:IN, :], preferred_element_type=jnp.float32)
            + jnp.dot(pb, wg_ref[IN:, :], preferred_element_type=jnp.float32)
        )
        yp_ref[...] = jnp.dot(pb, wp_ref[...], preferred_element_type=jnp.float32)

    return pl.pallas_call(
        body,
        grid=(N // BLK,),
        in_specs=[
            pl.BlockSpec((BLK, IN), lambda i: (i, 0)),
            pl.BlockSpec((BLK, pos.shape[1]), lambda i: (i, 0)),
            pl.BlockSpec(Wg1.shape, lambda i: (0, 0)),
            pl.BlockSpec(Wp1.shape, lambda i: (0, 0)),
        ],
        out_specs=[
            pl.BlockSpec((BLK, EMB), lambda i: (i, 0)),
            pl.BlockSpec((BLK, EMB), lambda i: (i, 0)),
        ],
        out_shape=[
            jax.ShapeDtypeStruct((N, EMB), jnp.float32),
            jax.ShapeDtypeStruct((N, EMB), jnp.float32),
        ],
    )(x, pos, Wg1, Wp1)


def _segment_sums(yg, yp, src, dst):
    N, D = yg.shape
    E = src.shape[0]
    L = 16
    K = 2000
    NBLK = E // K
    G = 64
    CAP = 4608
    THRESH = CAP - K
    TRASH = CAP + G
    ROWS_PT = (N // _NS) // 8 * 8
    REM = N - ROWS_PT * _NS
    AR = ROWS_PT + REM
    GR = AR

    mesh = plsc.VectorSubcoreMesh(core_axis_name="c", subcore_axis_name="s")

    @functools.partial(
        pl.kernel,
        mesh=mesh,
        compiler_params=pltpu.CompilerParams(needs_layout_passes=False),
        out_type=[
            jax.ShapeDtypeStruct((N, D), jnp.float32),
            jax.ShapeDtypeStruct((N, D), jnp.float32),
        ],
        scratch_types=[
            pltpu.VMEM((K,), jnp.int32),
            pltpu.VMEM((K,), jnp.int32),
            pltpu.VMEM((K,), jnp.int32),
            pltpu.VMEM((K,), jnp.int32),
            pltpu.VMEM((CAP + G + 8,), jnp.int32),
            pltpu.VMEM((CAP + G + 8,), jnp.int32),
            pltpu.VMEM((G, D), jnp.float32),
            pltpu.VMEM((G, D), jnp.float32),
            pltpu.VMEM((AR + 8, D), jnp.float32),
            pltpu.SemaphoreType.DMA,
            pltpu.SemaphoreType.DMA,
        ],
    )
    def sck(yg_hbm, yp_hbm, src_hbm, dst_hbm, og_hbm, op_hbm,
            srca_v, srcb_v, dsta_v, dstb_v, csrc_v, cdst_v,
            rowsa_v, rowsb_v, acc_v, sem_i, sem_g):
        cid = lax.axis_index("c")
        sid = lax.axis_index("s")
        lo = sid * ROWS_PT
        hi = jnp.where(sid == _NS - 1, N, lo + ROWS_PT)
        lov = jnp.full((L,), lo, jnp.int32)
        hiv = jnp.full((L,), hi, jnp.int32)
        trashv = jnp.full((L,), TRASH, jnp.int32)

        def zero_row(r, carry):
            for cc in range(D // L):
                acc_v[r, pl.ds(cc * L, L)] = jnp.zeros((L,), jnp.float32)
            return carry

        lax.fori_loop(0, AR + 8, zero_row, 0)

        def issue_idx(b, sbuf, dbuf):
            pltpu.async_copy(src_hbm.at[pl.ds(b * K, K)], sbuf, sem_i)
            pltpu.async_copy(dst_hbm.at[pl.ds(b * K, K)], dbuf, sem_i)

        def wait_idx():
            pltpu.make_async_copy(src_hbm.at[pl.ds(0, K)], srca_v, sem_i).wait()
            pltpu.make_async_copy(dst_hbm.at[pl.ds(0, K)], srca_v, sem_i).wait()

        def run(y_hbm, out_hbm):
            def issue_g(bb, rbuf):
                pltpu.async_copy(
                    y_hbm.at[csrc_v.at[pl.ds(bb * G, G)]], rbuf, sem_g
                )

            def wait_g():
                pltpu.make_async_copy(
                    y_hbm.at[pl.ds(0, G)], rowsa_v, sem_g
                ).wait()

            def accum(bb, rbuf):
                def grp(j, c2):
                    dv = cdst_v[pl.ds(bb * G + j * L, L)]
                    for jj in range(L):
                        dloc = dv[jj]
                        e = j * L + jj
                        vals = [rbuf[e, pl.ds(cc * L, L)]
                                for cc in range(D // L)]
                        for cc in range(D // L):
                            plsc.addupdate(
                                acc_v.at[dloc, pl.ds(cc * L, L)], vals[cc]
                            )
                    return c2

                lax.fori_loop(0, G // L, grp, 0)

            def drain(cnt):
                for jj in range(G // L):
                    csrc_v[pl.ds(cnt + jj * L, L)] = jnp.zeros((L,), jnp.int32)
                    cdst_v[pl.ds(cnt + jj * L, L)] = jnp.full((L,), GR, jnp.int32)
                nb = (cnt + G - 1) // G

                @pl.when(nb > 0)
                def _():
                    issue_g(0, rowsa_v)

                def pair(i, carry):
                    bb = 2 * i
                    wait_g()

                    @pl.when(bb + 1 < nb)
                    def _():
                        issue_g(bb + 1, rowsb_v)

                    accum(bb, rowsa_v)

                    @pl.when(bb + 1 < nb)
                    def _():
                        wait_g()

                        @pl.when(bb + 2 < nb)
                        def _():
                            issue_g(bb + 2, rowsa_v)

                        accum(bb + 1, rowsb_v)

                    return carry

                lax.fori_loop(0, (nb + 1) // 2, pair, 0)

            def filter_block(sbuf, dbuf, cntv):
                def filt(j, cv):
                    d = dbuf[pl.ds(j * L, L)]
                    sv = sbuf[pl.ds(j * L, L)]
                    m = (d >= lov) & (d < hiv)
                    cum = plsc.cumsum(jnp.where(m, 1, 0))
                    pos = jnp.where(m, cv + cum - 1, trashv)
                    plsc.store_scatter(csrc_v, [pos], sv)
                    plsc.store_scatter(cdst_v, [pos], d - lov)
                    return cv + plsc.all_reduce_population_count(m)

                cntv = lax.fori_loop(0, K // L, filt, cntv)
                cnt = cntv[0]

                @pl.when(cnt >= THRESH)
                def _():
                    drain(cnt)

                return jnp.where(cnt >= THRESH, jnp.zeros_like(cntv), cntv)

            def blockpair(i, cntv):
                b = 2 * i
                wait_idx()

                @pl.when(b + 1 < NBLK)
                def _():
                    issue_idx(b + 1, srcb_v, dstb_v)

                cntv = filter_block(srca_v, dsta_v, cntv)

                @pl.when(b + 1 < NBLK)
                def _2():
                    wait_idx()

                    @pl.when(b + 2 < NBLK)
                    def _():
                        issue_idx(b + 2, srca_v, dsta_v)

                cntv = filter_block(srcb_v, dstb_v, cntv)
                return cntv

            cntv = lax.fori_loop(0, (NBLK + 1) // 2,
                                 blockpair, jnp.zeros((L,), jnp.int32))
            drain(cntv[0])

            pltpu.sync_copy(
                acc_v.at[pl.ds(0, ROWS_PT)],
                out_hbm.at[pl.ds(sid * ROWS_PT, ROWS_PT)],
            )

            @pl.when(sid == _NS - 1)
            def _():
                pltpu.sync_copy(
                    acc_v.at[pl.ds(ROWS_PT, REM)],
                    out_hbm.at[pl.ds(_NS * ROWS_PT, REM)],
                )

        issue_idx(0, srca_v, dsta_v)

        @pl.when(cid == 0)
        def _():
            run(yg_hbm, og_hbm)

        @pl.when(cid == 1)
        def _():
            run(yp_hbm, op_hbm)

    return sck(yg, yp, src, dst)


def _ln_rows(h, g, b):
    mu = jnp.mean(h, axis=-1, keepdims=True)
    var = jnp.mean((h - mu) * (h - mu), axis=-1, keepdims=True)
    return (h - mu) * jax.lax.rsqrt(var + 1e-5) * g + b


def _epilogue(yg, agg_g, yp, agg_p, x, sg, sp, bg1, lng_g, lng_b, Wg2, bg2,
              bn_g, bn_b, bp1, lnp_g, lnp_b, Wp2, bp2, bnp_g, bnp_b):
    N, D = yg.shape
    BLK = 1000

    def body(sg_ref, sp_ref, yg_ref, ag_ref, yp_ref, ap_ref, x_ref,
             bg1_ref, lng_g_ref, lng_b_ref, wg2_ref, bg2_ref, bn_g_ref, bn_b_ref,
             bp1_ref, lnp_g_ref, lnp_b_ref, wp2_ref, bp2_ref, bnp_g_ref, bnp_b_ref,
             h_ref, p_ref):
        hg = sg_ref[0, 0] * yg_ref[...] + ag_ref[...] + bg1_ref[...]
        hg = jax.nn.relu(_ln_rows(hg, lng_g_ref[...], lng_b_ref[...]))
        hg = jnp.dot(hg, wg2_ref[...], preferred_element_type=jnp.float32) + bg2_ref[...]
        hg = jax.nn.relu(_ln_rows(hg, bn_g_ref[...], bn_b_ref[...]))
        h_ref[...] = hg + x_ref[...]

        hp = sp_ref[0, 0] * yp_ref[...] + ap_ref[...] + bp1_ref[...]
        hp = jax.nn.relu(_ln_rows(hp, lnp_g_ref[...], lnp_b_ref[...]))
        hp = jnp.dot(hp, wp2_ref[...], preferred_element_type=jnp.float32) + bp2_ref[...]
        p_ref[...] = jax.nn.relu(_ln_rows(hp, bnp_g_ref[...], bnp_b_ref[...]))

    row = lambda a: a.reshape(1, D)
    vec_spec = pl.BlockSpec((1, D), lambda i: (0, 0))
    blk_spec = pl.BlockSpec((BLK, D), lambda i: (i, 0))
    mat_spec = pl.BlockSpec((D, D), lambda i: (0, 0))
    smem_spec = pl.BlockSpec(memory_space=pltpu.SMEM)

    return pl.pallas_call(
        body,
        grid=(N // BLK,),
        in_specs=[smem_spec, smem_spec,
                  blk_spec, blk_spec, blk_spec, blk_spec, blk_spec,
                  vec_spec, vec_spec, vec_spec, mat_spec, vec_spec, vec_spec, vec_spec,
                  vec_spec, vec_spec, vec_spec, mat_spec, vec_spec, vec_spec, vec_spec],
        out_specs=[blk_spec, blk_spec],
        out_shape=[
            jax.ShapeDtypeStruct((N, D), jnp.float32),
            jax.ShapeDtypeStruct((N, D), jnp.float32),
        ],
    )(sg.reshape(1, 1), sp.reshape(1, 1),
      yg, agg_g, yp, agg_p, x,
      row(bg1), row(lng_g), row(lng_b), Wg2, row(bg2), row(bn_g), row(bn_b),
      row(bp1), row(lnp_g), row(lnp_b), Wp2, row(bp2), row(bnp_g), row(bnp_b))


def kernel(x, pos_embeddings, eps_g, Wg1, bg1, lng_g, lng_b, Wg2, bg2, bn_g,
           bn_b, eps_p, Wp1, bp1, lnp_g, lnp_b, Wp2, bp2, bnp_g, bnp_b,
           edge_index):
    yg, yp = _pre_project(x, pos_embeddings, Wg1, Wp1)
    src = edge_index[0]
    dst = edge_index[1]
    agg_g, agg_p = _segment_sums(yg, yp, src, dst)
    h, p = _epilogue(yg, agg_g, yp, agg_p, x,
                     1.0 + eps_g, 1.0 + eps_p,
                     bg1, lng_g, lng_b, Wg2, bg2, bn_g, bn_b,
                     bp1, lnp_g, lnp_b, Wp2, bp2, bnp_g, bnp_b)
    return (h, p)
